# Initial kernel scaffold; baseline (speedup 1.0000x reference)
import jax
import jax.numpy as jnp
from jax.experimental import pallas as pl


def kernel(x, labels):
    raise NotImplementedError("write your pallas kernel here")


# speedup vs baseline: 7.9931x; 7.9931x over previous
import jax
import jax.numpy as jnp
from jax.experimental import pallas as pl
from jax.experimental.pallas import tpu as pltpu

K = 100
NB = 100


def _pass1(lab_ref, x_ref, sums_ref, counts_ref):
    i = pl.program_id(0)
    lab = lab_ref[0, 0, :]
    b = lab.shape[0]
    oh = (jax.lax.broadcasted_iota(jnp.int32, (K, b), 0) == lab[None, :]
          ).astype(jnp.float32)
    part = jax.lax.dot_general(
        oh, x_ref[...], (((1,), (0,)), ((), ())),
        preferred_element_type=jnp.float32)
    cnt = jnp.sum(oh, axis=1)

    @pl.when(i == 0)
    def _():
        sums_ref[...] = jnp.zeros_like(sums_ref)
        counts_ref[...] = jnp.zeros_like(counts_ref)

    sums_ref[...] += part
    counts_ref[...] += jnp.broadcast_to(cnt[:, None], counts_ref.shape)


def _pass2(lab_ref, x_ref, sums_ref, counts_ref, loss_ref, centers_ref, acc_ref):
    i = pl.program_id(0)

    @pl.when(i == 0)
    def _():
        centers_ref[...] = sums_ref[...] / counts_ref[...]
        acc_ref[0] = 0.0

    lab = lab_ref[0, 0, :]
    b = lab.shape[0]
    oh = (jax.lax.broadcasted_iota(jnp.int32, (K, b), 0) == lab[None, :]
          ).astype(jnp.float32)
    ch = jax.lax.dot_general(
        oh, centers_ref[...], (((0,), (0,)), ((), ())),
        preferred_element_type=jnp.float32)
    d = x_ref[...] - ch
    rs = jnp.sum(d * d, axis=1)
    acc_ref[0] += jnp.sum(jnp.sqrt(rs))

    @pl.when(i == pl.num_programs(0) - 1)
    def _():
        c = centers_ref[...]
        n2 = jnp.sum(c * c, axis=1)
        g = jax.lax.dot_general(
            c, c, (((1,), (1,)), ((), ())), preferred_element_type=jnp.float32)
        d2 = jnp.maximum(n2[:, None] + n2[None, :] - 2.0 * g, 0.0)
        dm = jnp.sqrt(d2)
        ii = jax.lax.broadcasted_iota(jnp.int32, (K, K), 0)
        jj = jax.lax.broadcasted_iota(jnp.int32, (K, K), 1)
        inter = jnp.sum(jnp.where(jj > ii, dm, 0.0))
        intra = acc_ref[0]
        loss_ref[0, 0] = jnp.where(inter > 0, intra / inter, intra)


def kernel(x, labels):
    n, d_model = x.shape
    b = n // NB
    lab3 = labels.astype(jnp.int32).reshape(NB, 1, b)

    sums, counts = pl.pallas_call(
        _pass1,
        grid=(NB,),
        in_specs=[
            pl.BlockSpec((1, 1, b), lambda i: (i, 0, 0)),
            pl.BlockSpec((b, d_model), lambda i: (i, 0)),
        ],
        out_specs=[
            pl.BlockSpec((K, d_model), lambda i: (0, 0)),
            pl.BlockSpec((K, d_model), lambda i: (0, 0)),
        ],
        out_shape=[
            jax.ShapeDtypeStruct((K, d_model), jnp.float32),
            jax.ShapeDtypeStruct((K, d_model), jnp.float32),
        ],
        compiler_params=pltpu.CompilerParams(
            dimension_semantics=("arbitrary",)),
    )(lab3, x)

    loss = pl.pallas_call(
        _pass2,
        grid=(NB,),
        in_specs=[
            pl.BlockSpec((1, 1, b), lambda i: (i, 0, 0)),
            pl.BlockSpec((b, d_model), lambda i: (i, 0)),
            pl.BlockSpec((K, d_model), lambda i: (0, 0)),
            pl.BlockSpec((K, d_model), lambda i: (0, 0)),
        ],
        out_specs=pl.BlockSpec(memory_space=pltpu.SMEM),
        out_shape=jax.ShapeDtypeStruct((1, 1), jnp.float32),
        scratch_shapes=[
            pltpu.VMEM((K, d_model), jnp.float32),
            pltpu.SMEM((1,), jnp.float32),
        ],
        compiler_params=pltpu.CompilerParams(
            dimension_semantics=("arbitrary",)),
    )(lab3, x, sums, counts)

    return loss[0, 0]
